# Initial kernel scaffold; baseline (speedup 1.0000x reference)
import jax
import jax.numpy as jnp
from jax.experimental import pallas as pl


def kernel(x, dataset_token, coors, bn_gamma, bn_beta, W, b):
    raise NotImplementedError("write your pallas kernel here")


# speedup vs baseline: 4.8020x; 4.8020x over previous
import functools

import jax
import jax.numpy as jnp
from jax import lax
from jax.experimental import pallas as pl
from jax.experimental.pallas import tpu as pltpu

N, C, B, CTX = 100000, 256, 4, 256
RB = 2000
NB = N // RB


def _stats_body(x_ref, out_ref):
    @pl.when(pl.program_id(0) == 0)
    def _():
        out_ref[...] = jnp.zeros_like(out_ref)

    xv = x_ref[...]
    s = jnp.sum(xv, axis=0, keepdims=True)
    sq = jnp.sum(xv * xv, axis=0, keepdims=True)
    out_ref[0:1, :] += s
    out_ref[1:2, :] += sq


def _apply_body(sums_ref, tok_ref, w_ref, b_ref, gamma_ref, beta_ref,
                bidx_ref, x_ref, out_ref, ad_ref):
    @pl.when(pl.program_id(0) == 0)
    def _():
        mean = sums_ref[0:1, :] / N
        var = sums_ref[1:2, :] / N - mean * mean
        g = gamma_ref[...] * lax.rsqrt(var + 1e-5)
        base_shift = beta_ref[...] - mean * g
        tok = tok_ref[...]
        h = tok * (1.0 / (1.0 + jnp.exp(-tok)))
        sc = lax.dot_general(h, w_ref[...],
                             (((1,), (1,)), ((), ())),
                             preferred_element_type=jnp.float32)
        sc = sc + b_ref[...]
        shift = sc[:, :C]
        scale = sc[:, C:]
        one_p = 1.0 + scale
        ad_ref[0:B, :] = one_p * g
        ad_ref[B:2 * B, :] = one_p * base_shift + shift

    bidx = bidx_ref[0, 0, :]
    oh = (bidx[:, None] ==
          lax.broadcasted_iota(jnp.int32, (RB, B), 1)).astype(jnp.float32)
    a_rows = jnp.dot(oh, ad_ref[0:B, :],
                     preferred_element_type=jnp.float32)
    d_rows = jnp.dot(oh, ad_ref[B:2 * B, :],
                     preferred_element_type=jnp.float32)
    out_ref[...] = x_ref[...] * a_rows + d_rows


@jax.jit
def kernel(x, dataset_token, coors, bn_gamma, bn_beta, W, b):
    sums = pl.pallas_call(
        _stats_body,
        grid=(NB,),
        in_specs=[pl.BlockSpec((RB, C), lambda i: (i, 0))],
        out_specs=pl.BlockSpec((8, C), lambda i: (0, 0)),
        out_shape=jax.ShapeDtypeStruct((8, C), jnp.float32),
    )(x)

    bidx3 = coors.reshape(NB, 1, RB)
    gamma2 = bn_gamma.reshape(1, C)
    beta2 = bn_beta.reshape(1, C)
    b2 = b.reshape(1, 2 * C)

    out = pl.pallas_call(
        _apply_body,
        grid=(NB,),
        in_specs=[
            pl.BlockSpec((8, C), lambda i: (0, 0)),
            pl.BlockSpec((B, CTX), lambda i: (0, 0)),
            pl.BlockSpec((2 * C, CTX), lambda i: (0, 0)),
            pl.BlockSpec((1, 2 * C), lambda i: (0, 0)),
            pl.BlockSpec((1, C), lambda i: (0, 0)),
            pl.BlockSpec((1, C), lambda i: (0, 0)),
            pl.BlockSpec((1, 1, RB), lambda i: (i, 0, 0)),
            pl.BlockSpec((RB, C), lambda i: (i, 0)),
        ],
        out_specs=pl.BlockSpec((RB, C), lambda i: (i, 0)),
        out_shape=jax.ShapeDtypeStruct((N, C), jnp.float32),
        scratch_shapes=[pltpu.VMEM((2 * B, C), jnp.float32)],
    )(sums, dataset_token, W, b2, gamma2, beta2, bidx3, x)
    return out
